# baseline (speedup 1.0000x reference)
import functools

import jax
import jax.numpy as jnp
from jax import lax
from jax.experimental import pallas as pl
from jax.experimental.pallas import tpu as pltpu
from jax.experimental.pallas import tpu_sc as plsc

_NEG = -1e9


def _emb_gather(table, ids):
    B = ids.shape[0]
    D = table.shape[1]
    info = plsc.get_sparse_core_info()
    nw = info.num_cores * info.num_subcores
    b_per_w = B // nw
    ch = 128
    n_ch = b_per_w // ch
    mesh = plsc.VectorSubcoreMesh(core_axis_name="c", subcore_axis_name="s")

    @functools.partial(
        pl.kernel,
        mesh=mesh,
        out_type=jax.ShapeDtypeStruct((B, D), jnp.float32),
        scratch_types=[
            pltpu.VMEM((ch,), jnp.int32),
            pltpu.VMEM((ch,), jnp.int32),
            pltpu.VMEM((ch, D), jnp.float32),
            pltpu.VMEM((ch, D), jnp.float32),
            pltpu.SemaphoreType.DMA,
            pltpu.SemaphoreType.DMA,
        ],
    )
    def gather_k(table_hbm, idx_hbm, out_hbm, idx0, idx1, rows0, rows1, sem0, sem1):
        wid = lax.axis_index("s") * info.num_cores + lax.axis_index("c")
        base = wid * b_per_w
        idx_v = (idx0, idx1)
        rows_v = (rows0, rows1)
        sems = (sem0, sem1)
        pltpu.sync_copy(idx_hbm.at[pl.ds(base, ch)], idx0)
        copy0 = pltpu.async_copy(table_hbm.at[idx0], rows0, sem0)
        for j in range(n_ch):
            cur = j % 2
            nxt = (j + 1) % 2
            if j + 1 < n_ch:
                pltpu.sync_copy(idx_hbm.at[pl.ds(base + (j + 1) * ch, ch)], idx_v[nxt])
                pltpu.async_copy(table_hbm.at[idx_v[nxt]], rows_v[nxt], sems[nxt])
            pltpu.make_async_copy(table_hbm.at[idx_v[cur]], rows_v[cur], sems[cur]).wait()
            pltpu.sync_copy(rows_v[cur], out_hbm.at[pl.ds(base + j * ch, ch)])
        del copy0

    return gather_k(table, ids)


def _pad_gates(wt, H, hp):
    return jnp.concatenate(
        [jnp.pad(wt[:, i * H:(i + 1) * H], ((0, 0), (0, hp - H)))
         for i in range(3)], axis=1)


def _pad_rows(w, H, hp):
    return jnp.concatenate(
        [jnp.pad(w[i * H:(i + 1) * H], ((0, hp - H), (0, 0)))
         for i in range(2)], axis=0)


def _word_body(T, HP, nc, x_ref, lr_ref, lc_ref,
               wfx, bfx, wbx, bbx, whh, bhh, aw, ab, ac,
               out_ref, gxf_ref, gxb_ref, hall_ref):
    E = x_ref.shape[-1]
    G = 3 * HP
    x2 = x_ref[...].reshape(T * nc, E).astype(jnp.bfloat16)
    gxf_ref[...] = (jnp.dot(x2, wfx[...].astype(jnp.bfloat16), preferred_element_type=jnp.float32)
                    + bfx[...]).reshape(T, nc, G)
    gxb_ref[...] = (jnp.dot(x2, wbx[...].astype(jnp.bfloat16), preferred_element_type=jnp.float32)
                    + bbx[...]).reshape(T, nc, G)
    lens_c = lc_ref[:, 0:1]

    whh_v = whh[...].astype(jnp.bfloat16)
    bhh_v = bhh[...]

    def step(u, hcat):
        gh = jnp.dot(hcat.astype(jnp.bfloat16), whh_v, preferred_element_type=jnp.float32) + bhh_v
        gf = gxf_ref[u]
        gb = gxb_ref[T - 1 - u]
        rf = jax.nn.sigmoid(gf[:, 0:HP] + gh[:, 0:HP])
        zf = jax.nn.sigmoid(gf[:, HP:2 * HP] + gh[:, HP:2 * HP])
        nf = jnp.tanh(gf[:, 2 * HP:G] + rf * gh[:, 2 * HP:G])
        rb = jax.nn.sigmoid(gb[:, 0:HP] + gh[:, G:G + HP])
        zb = jax.nn.sigmoid(gb[:, HP:2 * HP] + gh[:, G + HP:G + 2 * HP])
        nb = jnp.tanh(gb[:, 2 * HP:G] + rb * gh[:, G + 2 * HP:2 * G])
        hf = (1.0 - zf) * nf + zf * hcat[:, 0:HP]
        hb = (1.0 - zb) * nb + zb * hcat[:, HP:2 * HP]
        hf = jnp.where(u < lens_c, hf, hcat[:, 0:HP])
        hb = jnp.where(T - 1 - u < lens_c, hb, hcat[:, HP:2 * HP])
        hall_ref[u, :, 0:HP] = hf
        hall_ref[T - 1 - u, :, HP:2 * HP] = hb
        return jnp.concatenate([hf, hb], axis=1)

    lax.fori_loop(0, T, step, jnp.zeros((nc, 2 * HP), jnp.float32))

    hall = hall_ref[...]
    A = aw.shape[-1]
    a = jnp.tanh(jnp.dot(hall.reshape(T * nc, 2 * HP), aw[...],
                         preferred_element_type=jnp.float32) + ab[...])
    s = jnp.sum(a.reshape(T, nc, A) * ac[...].reshape(1, 1, A), axis=-1)
    lens_r = lr_ref[0:1, :]
    tmask = lax.broadcasted_iota(jnp.int32, (T, nc), 0) < lens_r
    s = jnp.where(tmask, s, _NEG)
    smax = jnp.max(s, axis=0, keepdims=True)
    e = jnp.exp(s - smax)
    al = e / jnp.sum(e, axis=0, keepdims=True)
    out_ref[...] = jnp.sum(hall * al[:, :, None], axis=0)


def _bigru_attend(x_tm, lens, pf, pb, aw, ab, ac, nc, hp):
    T, N, E = x_tm.shape
    H = pf['Whh'].shape[1]
    grid = N // nc
    lens_rows = jnp.broadcast_to(lens[None, :], (8, N))
    lens_cols = jnp.broadcast_to(lens[:, None], (N, 8))
    wfx = _pad_gates(pf['Wih'].T, H, hp)
    bfx = _pad_gates(pf['bih'].reshape(1, -1), H, hp)
    wbx = _pad_gates(pb['Wih'].T, H, hp)
    bbx = _pad_gates(pb['bih'].reshape(1, -1), H, hp)
    whf = jnp.pad(_pad_gates(pf['Whh'].T, H, hp), ((0, hp - H), (0, 0)))
    whb = jnp.pad(_pad_gates(pb['Whh'].T, H, hp), ((0, hp - H), (0, 0)))
    zer = jnp.zeros_like(whf)
    whh = jnp.concatenate(
        [jnp.concatenate([whf, zer], axis=1),
         jnp.concatenate([zer, whb], axis=1)], axis=0)
    bhh = jnp.concatenate([_pad_gates(pf['bhh'].reshape(1, -1), H, hp),
                           _pad_gates(pb['bhh'].reshape(1, -1), H, hp)], axis=1)
    awp = _pad_rows(aw, H, hp)
    ab2 = ab.reshape(1, -1)
    ac2 = ac.reshape(1, -1)

    def rep(shape):
        nd = len(shape)
        return pl.BlockSpec(shape, lambda i: (0,) * nd)

    return pl.pallas_call(
        functools.partial(_word_body, T, hp, nc),
        grid=(grid,),
        in_specs=[
            pl.BlockSpec((T, nc, E), lambda i: (0, i, 0)),
            pl.BlockSpec((8, nc), lambda i: (0, i)),
            pl.BlockSpec((nc, 8), lambda i: (i, 0)),
            rep(wfx.shape), rep(bfx.shape), rep(wbx.shape), rep(bbx.shape),
            rep(whh.shape), rep(bhh.shape),
            rep(awp.shape), rep(ab2.shape), rep(ac2.shape),
        ],
        out_specs=pl.BlockSpec((nc, 2 * hp), lambda i: (i, 0)),
        out_shape=jax.ShapeDtypeStruct((N, 2 * hp), jnp.float32),
        scratch_shapes=[
            pltpu.VMEM((T, nc, 3 * hp), jnp.float32),
            pltpu.VMEM((T, nc, 3 * hp), jnp.float32),
            pltpu.VMEM((T, nc, 2 * hp), jnp.float32),
        ],
    )(x_tm, lens_rows, lens_cols, wfx, bfx, wbx, bbx, whh, bhh,
      awp, ab2, ac2)


def _gru_gate(H, gx, gh, h):
    r = jax.nn.sigmoid(gx[:, :H] + gh[:, :H])
    z = jax.nn.sigmoid(gx[:, H:2 * H] + gh[:, H:2 * H])
    n = jnp.tanh(gx[:, 2 * H:] + r * gh[:, 2 * H:])
    return (1.0 - z) * n + z * h


def _attend_rows(hall, lens_c, aw, ab, ac):
    N, T, H2 = hall.shape
    A = aw.shape[-1]
    a = jnp.tanh(jnp.dot(hall.reshape(N * T, H2), aw[...],
                         preferred_element_type=jnp.float32) + ab[...])
    s = jnp.sum(a.reshape(N, T, A) * ac[...].reshape(1, 1, A), axis=-1)
    tmask = lax.broadcasted_iota(jnp.int32, (N, T), 1) < lens_c
    s = jnp.where(tmask, s, _NEG)
    e = jnp.exp(s - jnp.max(s, axis=1, keepdims=True))
    al = e / jnp.sum(e, axis=1, keepdims=True)
    return jnp.sum(hall * al[:, :, None], axis=1)


def _upper_body(sx_ref, slc_ref, plc_ref,
                wsfx, bsfx, wsfh, bsfh, wsbx, bsbx, wsbh, bsbh, saw, sab, sac,
                wpfx, bpfx, wpfh, bpfh, wpbx, bpbx, wpbh, bpbh, paw, pab, pac,
                w1, b1, w2, b2, out_ref,
                gxsf_ref, gxsb_ref, halls_ref, preps_ref, hallp_ref):
    NJ, TS, ES = sx_ref.shape
    HS = 200
    HP = 300
    NK = 16
    TP = 4
    slc = slc_ref[:, 0:1]
    plc = plc_ref[:, 0:1]

    x2 = sx_ref[...].reshape(NJ * TS, ES)
    gxsf_ref[...] = (jnp.dot(x2, wsfx[...], preferred_element_type=jnp.float32)
                     + bsfx[...]).reshape(NJ, TS, 3 * HS)
    gxsb_ref[...] = (jnp.dot(x2, wsbx[...], preferred_element_type=jnp.float32)
                     + bsbx[...]).reshape(NJ, TS, 3 * HS)
    hf = jnp.zeros((NJ, HS), jnp.float32)
    hb = jnp.zeros((NJ, HS), jnp.float32)
    for t in range(TS):
        ghf = jnp.dot(hf, wsfh[...], preferred_element_type=jnp.float32) + bsfh[...]
        ghb = jnp.dot(hb, wsbh[...], preferred_element_type=jnp.float32) + bsbh[...]
        nhf = _gru_gate(HS, gxsf_ref[:, t, :], ghf, hf)
        nhb = _gru_gate(HS, gxsb_ref[:, TS - 1 - t, :], ghb, hb)
        hf = jnp.where(t < slc, nhf, hf)
        hb = jnp.where(TS - 1 - t < slc, nhb, hb)
        halls_ref[:, t, 0:HS] = hf
        halls_ref[:, TS - 1 - t, HS:2 * HS] = hb
    preps_ref[...] = _attend_rows(halls_ref[...], slc, saw, sab, sac)

    hf = jnp.zeros((NK, HP), jnp.float32)
    hb = jnp.zeros((NK, HP), jnp.float32)
    for t in range(TP):
        xf = preps_ref[t * NK:(t + 1) * NK]
        xb = preps_ref[(TP - 1 - t) * NK:(TP - t) * NK]
        gf = jnp.dot(xf, wpfx[...], preferred_element_type=jnp.float32) + bpfx[...]
        gb = jnp.dot(xb, wpbx[...], preferred_element_type=jnp.float32) + bpbx[...]
        ghf = jnp.dot(hf, wpfh[...], preferred_element_type=jnp.float32) + bpfh[...]
        ghb = jnp.dot(hb, wpbh[...], preferred_element_type=jnp.float32) + bpbh[...]
        nhf = _gru_gate(HP, gf, ghf, hf)
        nhb = _gru_gate(HP, gb, ghb, hb)
        hf = jnp.where(t < plc, nhf, hf)
        hb = jnp.where(TP - 1 - t < plc, nhb, hb)
        hallp_ref[:, t, 0:HP] = hf
        hallp_ref[:, TP - 1 - t, HP:2 * HP] = hb
    docrep = _attend_rows(hallp_ref[...], plc, paw, pab, pac)

    cur = docrep[0:8]
    prev = docrep[8:16]
    cat = jnp.concatenate([cur, prev, jnp.abs(cur - prev)], axis=1)
    h = jax.nn.relu(jnp.dot(cat, w1[...], preferred_element_type=jnp.float32)
                    + b1[...])
    out_ref[...] = jax.nn.sigmoid(
        jnp.dot(h, w2[...], preferred_element_type=jnp.float32) + b2[...])


def _upper(sx, slens, plens, ps_f, ps_b, saw, sab, sac,
           pp_f, pp_b, paw, pab, pac, w1, b1, w2, b2, wh, whp):
    HS, HP = 200, 300
    slc = jnp.broadcast_to(slens[:, None], (slens.shape[0], 8))
    plc = jnp.broadcast_to(plens[:, None], (plens.shape[0], 8))
    return pl.pallas_call(
        _upper_body,
        out_shape=jax.ShapeDtypeStruct((8, 1), jnp.float32),
        scratch_shapes=[
            pltpu.VMEM((64, 16, 3 * HS), jnp.float32),
            pltpu.VMEM((64, 16, 3 * HS), jnp.float32),
            pltpu.VMEM((64, 16, 2 * HS), jnp.float32),
            pltpu.VMEM((64, 2 * HS), jnp.float32),
            pltpu.VMEM((16, 4, 2 * HP), jnp.float32),
        ],
    )(sx, slc, plc,
      _pad_rows(ps_f['Wih'].T, wh, whp), ps_f['bih'].reshape(1, -1), ps_f['Whh'].T, ps_f['bhh'].reshape(1, -1),
      _pad_rows(ps_b['Wih'].T, wh, whp), ps_b['bih'].reshape(1, -1), ps_b['Whh'].T, ps_b['bhh'].reshape(1, -1),
      saw, sab.reshape(1, -1), sac.reshape(1, -1),
      pp_f['Wih'].T, pp_f['bih'].reshape(1, -1), pp_f['Whh'].T, pp_f['bhh'].reshape(1, -1),
      pp_b['Wih'].T, pp_b['bih'].reshape(1, -1), pp_b['Whh'].T, pp_b['bhh'].reshape(1, -1),
      paw, pab.reshape(1, -1), pac.reshape(1, -1),
      w1, b1.reshape(1, -1), w2, b2.reshape(1, -1))


def kernel(current_document, words_per_sentence_current_document,
           sentences_per_paragraph_current_document,
           paragraphs_per_document_current_document, previous_document,
           words_per_sentence_previous_document,
           sentences_per_paragraph_previous_document,
           paragraphs_per_document_previous_document, click_rate_tensor,
           params):
    p = params
    B, P, S, W = current_document.shape
    EMB = p['emb'].shape[1]

    nw = 2 * B * P * S
    ids = jnp.concatenate([current_document.reshape(-1),
                           previous_document.reshape(-1)]).astype(jnp.int32)
    ids_tm = ids.reshape(2, B, P, S, W).transpose(4, 2, 0, 1, 3).reshape(-1)
    emb = _emb_gather(p['emb'], ids_tm)

    x_w = emb.reshape(W, nw, EMB)
    wlens = jnp.concatenate([
        words_per_sentence_current_document.reshape(-1),
        words_per_sentence_previous_document.reshape(-1)]).astype(jnp.int32)
    wlens = wlens.reshape(2, B, P, S).transpose(2, 0, 1, 3).reshape(-1)
    WH = p['word_f']['Whh'].shape[1]
    WHP = 128
    sreps = _bigru_attend(x_w, wlens, p['word_f'], p['word_b'],
                          p['watt_W'], p['watt_b'], p['watt_c'],
                          nc=256, hp=WHP)

    sx = sreps.reshape(2 * B * P, S, sreps.shape[-1])
    slens = jnp.concatenate([
        sentences_per_paragraph_current_document.reshape(-1),
        sentences_per_paragraph_previous_document.reshape(-1)]).astype(jnp.int32)
    slens = slens.reshape(2, B, P).transpose(2, 0, 1).reshape(-1)
    plens = jnp.concatenate([
        paragraphs_per_document_current_document.reshape(-1),
        paragraphs_per_document_previous_document.reshape(-1)]).astype(jnp.int32)
    return _upper(sx, slens, plens, p['sent_f'], p['sent_b'],
                  p['satt_W'], p['satt_b'], p['satt_c'],
                  p['para_f'], p['para_b'],
                  p['patt_W'], p['patt_b'], p['patt_c'],
                  p['cls_W1'], p['cls_b1'], p['cls_W2'], p['cls_b2'],
                  WH, WHP)


# speedup vs baseline: 1.0664x; 1.0664x over previous
import functools

import jax
import jax.numpy as jnp
from jax import lax
from jax.experimental import pallas as pl
from jax.experimental.pallas import tpu as pltpu
from jax.experimental.pallas import tpu_sc as plsc

_NEG = -1e9


def _emb_gather(table, ids):
    B = ids.shape[0]
    D = table.shape[1]
    info = plsc.get_sparse_core_info()
    nw = info.num_cores * info.num_subcores
    b_per_w = B // nw
    ch = 128
    n_ch = b_per_w // ch
    mesh = plsc.VectorSubcoreMesh(core_axis_name="c", subcore_axis_name="s")

    @functools.partial(
        pl.kernel,
        mesh=mesh,
        out_type=jax.ShapeDtypeStruct((B, D), jnp.float32),
        scratch_types=[
            pltpu.VMEM((ch,), jnp.int32),
            pltpu.VMEM((ch,), jnp.int32),
            pltpu.VMEM((ch, D), jnp.float32),
            pltpu.VMEM((ch, D), jnp.float32),
            pltpu.SemaphoreType.DMA,
            pltpu.SemaphoreType.DMA,
        ],
    )
    def gather_k(table_hbm, idx_hbm, out_hbm, idx0, idx1, rows0, rows1, sem0, sem1):
        wid = lax.axis_index("s") * info.num_cores + lax.axis_index("c")
        base = wid * b_per_w
        idx_v = (idx0, idx1)
        rows_v = (rows0, rows1)
        sems = (sem0, sem1)
        pltpu.sync_copy(idx_hbm.at[pl.ds(base, ch)], idx0)
        copy0 = pltpu.async_copy(table_hbm.at[idx0], rows0, sem0)
        for j in range(n_ch):
            cur = j % 2
            nxt = (j + 1) % 2
            if j + 1 < n_ch:
                pltpu.sync_copy(idx_hbm.at[pl.ds(base + (j + 1) * ch, ch)], idx_v[nxt])
                pltpu.async_copy(table_hbm.at[idx_v[nxt]], rows_v[nxt], sems[nxt])
            pltpu.make_async_copy(table_hbm.at[idx_v[cur]], rows_v[cur], sems[cur]).wait()
            pltpu.sync_copy(rows_v[cur], out_hbm.at[pl.ds(base + j * ch, ch)])
        del copy0

    return gather_k(table, ids)


def _pad_gates(wt, H, hp):
    return jnp.concatenate(
        [jnp.pad(wt[:, i * H:(i + 1) * H], ((0, 0), (0, hp - H)))
         for i in range(3)], axis=1)


def _pad_rows(w, H, hp):
    return jnp.concatenate(
        [jnp.pad(w[i * H:(i + 1) * H], ((0, hp - H), (0, 0)))
         for i in range(2)], axis=0)


def _word_body(T, HP, nc, x_ref, lr_ref, lc_ref,
               wfx, bfx, wbx, bbx, whh, bhh, aw, ab, ac,
               out_ref, gxf_ref, gxb_ref, hall_ref):
    E = x_ref.shape[-1]
    G = 3 * HP
    x2 = x_ref[...].reshape(T * nc, E).astype(jnp.bfloat16)
    gxf_ref[...] = ((jnp.dot(x2, wfx[...].astype(jnp.bfloat16),
                             preferred_element_type=jnp.float32)
                     + bfx[...]).reshape(T, nc, G)).astype(jnp.bfloat16)
    gxb_ref[...] = ((jnp.dot(x2, wbx[...].astype(jnp.bfloat16),
                             preferred_element_type=jnp.float32)
                     + bbx[...]).reshape(T, nc, G)).astype(jnp.bfloat16)
    lens_c = lc_ref[:, 0:1]

    whh_v = whh[...].astype(jnp.bfloat16)
    bhh_v = bhh[...]

    def step(u, hcat):
        gh = jnp.dot(hcat.astype(jnp.bfloat16), whh_v, preferred_element_type=jnp.float32) + bhh_v
        gf = gxf_ref[u]
        gb = gxb_ref[T - 1 - u]
        rf = jax.nn.sigmoid(gf[:, 0:HP] + gh[:, 0:HP])
        zf = jax.nn.sigmoid(gf[:, HP:2 * HP] + gh[:, HP:2 * HP])
        nf = jnp.tanh(gf[:, 2 * HP:G] + rf * gh[:, 2 * HP:G])
        rb = jax.nn.sigmoid(gb[:, 0:HP] + gh[:, G:G + HP])
        zb = jax.nn.sigmoid(gb[:, HP:2 * HP] + gh[:, G + HP:G + 2 * HP])
        nb = jnp.tanh(gb[:, 2 * HP:G] + rb * gh[:, G + 2 * HP:2 * G])
        hf = (1.0 - zf) * nf + zf * hcat[:, 0:HP]
        hb = (1.0 - zb) * nb + zb * hcat[:, HP:2 * HP]
        hf = jnp.where(u < lens_c, hf, hcat[:, 0:HP])
        hb = jnp.where(T - 1 - u < lens_c, hb, hcat[:, HP:2 * HP])
        hall_ref[u, :, 0:HP] = hf.astype(jnp.bfloat16)
        hall_ref[T - 1 - u, :, HP:2 * HP] = hb.astype(jnp.bfloat16)
        return jnp.concatenate([hf, hb], axis=1)

    lax.fori_loop(0, T, step, jnp.zeros((nc, 2 * HP), jnp.float32))

    hall = hall_ref[...]
    A = aw.shape[-1]
    a = jnp.tanh(jnp.dot(hall.reshape(T * nc, 2 * HP),
                         aw[...].astype(jnp.bfloat16),
                         preferred_element_type=jnp.float32) + ab[...])
    s = jnp.sum(a.reshape(T, nc, A) * ac[...].reshape(1, 1, A), axis=-1)
    lens_r = lr_ref[0:1, :]
    tmask = lax.broadcasted_iota(jnp.int32, (T, nc), 0) < lens_r
    s = jnp.where(tmask, s, _NEG)
    smax = jnp.max(s, axis=0, keepdims=True)
    e = jnp.exp(s - smax)
    al = e / jnp.sum(e, axis=0, keepdims=True)
    out_ref[...] = jnp.sum(hall * al[:, :, None], axis=0)


def _bigru_attend(x_tm, lens, pf, pb, aw, ab, ac, nc, hp):
    T, N, E = x_tm.shape
    H = pf['Whh'].shape[1]
    grid = N // nc
    lens_rows = jnp.broadcast_to(lens[None, :], (8, N))
    lens_cols = jnp.broadcast_to(lens[:, None], (N, 8))
    wfx = _pad_gates(pf['Wih'].T, H, hp)
    bfx = _pad_gates(pf['bih'].reshape(1, -1), H, hp)
    wbx = _pad_gates(pb['Wih'].T, H, hp)
    bbx = _pad_gates(pb['bih'].reshape(1, -1), H, hp)
    whf = jnp.pad(_pad_gates(pf['Whh'].T, H, hp), ((0, hp - H), (0, 0)))
    whb = jnp.pad(_pad_gates(pb['Whh'].T, H, hp), ((0, hp - H), (0, 0)))
    zer = jnp.zeros_like(whf)
    whh = jnp.concatenate(
        [jnp.concatenate([whf, zer], axis=1),
         jnp.concatenate([zer, whb], axis=1)], axis=0)
    bhh = jnp.concatenate([_pad_gates(pf['bhh'].reshape(1, -1), H, hp),
                           _pad_gates(pb['bhh'].reshape(1, -1), H, hp)], axis=1)
    awp = _pad_rows(aw, H, hp)
    ab2 = ab.reshape(1, -1)
    ac2 = ac.reshape(1, -1)

    def rep(shape):
        nd = len(shape)
        return pl.BlockSpec(shape, lambda i: (0,) * nd)

    return pl.pallas_call(
        functools.partial(_word_body, T, hp, nc),
        grid=(grid,),
        in_specs=[
            pl.BlockSpec((T, nc, E), lambda i: (0, i, 0)),
            pl.BlockSpec((8, nc), lambda i: (0, i)),
            pl.BlockSpec((nc, 8), lambda i: (i, 0)),
            rep(wfx.shape), rep(bfx.shape), rep(wbx.shape), rep(bbx.shape),
            rep(whh.shape), rep(bhh.shape),
            rep(awp.shape), rep(ab2.shape), rep(ac2.shape),
        ],
        out_specs=pl.BlockSpec((nc, 2 * hp), lambda i: (i, 0)),
        out_shape=jax.ShapeDtypeStruct((N, 2 * hp), jnp.float32),
        scratch_shapes=[
            pltpu.VMEM((T, nc, 3 * hp), jnp.bfloat16),
            pltpu.VMEM((T, nc, 3 * hp), jnp.bfloat16),
            pltpu.VMEM((T, nc, 2 * hp), jnp.bfloat16),
        ],
    )(x_tm, lens_rows, lens_cols, wfx, bfx, wbx, bbx, whh, bhh,
      awp, ab2, ac2)


def _gru_gate(H, gx, gh, h):
    r = jax.nn.sigmoid(gx[:, :H] + gh[:, :H])
    z = jax.nn.sigmoid(gx[:, H:2 * H] + gh[:, H:2 * H])
    n = jnp.tanh(gx[:, 2 * H:] + r * gh[:, 2 * H:])
    return (1.0 - z) * n + z * h


def _attend_rows(hall, lens_c, aw, ab, ac):
    N, T, H2 = hall.shape
    A = aw.shape[-1]
    a = jnp.tanh(jnp.dot(hall.reshape(N * T, H2), aw[...],
                         preferred_element_type=jnp.float32) + ab[...])
    s = jnp.sum(a.reshape(N, T, A) * ac[...].reshape(1, 1, A), axis=-1)
    tmask = lax.broadcasted_iota(jnp.int32, (N, T), 1) < lens_c
    s = jnp.where(tmask, s, _NEG)
    e = jnp.exp(s - jnp.max(s, axis=1, keepdims=True))
    al = e / jnp.sum(e, axis=1, keepdims=True)
    return jnp.sum(hall * al[:, :, None], axis=1)


def _upper_body(sx_ref, slc_ref, plc_ref,
                wsfx, bsfx, wsfh, bsfh, wsbx, bsbx, wsbh, bsbh, saw, sab, sac,
                wpfx, bpfx, wpfh, bpfh, wpbx, bpbx, wpbh, bpbh, paw, pab, pac,
                w1, b1, w2, b2, out_ref,
                gxsf_ref, gxsb_ref, halls_ref, preps_ref, hallp_ref):
    NJ, TS, ES = sx_ref.shape
    HS = 200
    HP = 300
    NK = 16
    TP = 4
    slc = slc_ref[:, 0:1]
    plc = plc_ref[:, 0:1]

    x2 = sx_ref[...].reshape(NJ * TS, ES)
    gxsf_ref[...] = (jnp.dot(x2, wsfx[...], preferred_element_type=jnp.float32)
                     + bsfx[...]).reshape(NJ, TS, 3 * HS)
    gxsb_ref[...] = (jnp.dot(x2, wsbx[...], preferred_element_type=jnp.float32)
                     + bsbx[...]).reshape(NJ, TS, 3 * HS)
    hf = jnp.zeros((NJ, HS), jnp.float32)
    hb = jnp.zeros((NJ, HS), jnp.float32)
    for t in range(TS):
        ghf = jnp.dot(hf, wsfh[...], preferred_element_type=jnp.float32) + bsfh[...]
        ghb = jnp.dot(hb, wsbh[...], preferred_element_type=jnp.float32) + bsbh[...]
        nhf = _gru_gate(HS, gxsf_ref[:, t, :], ghf, hf)
        nhb = _gru_gate(HS, gxsb_ref[:, TS - 1 - t, :], ghb, hb)
        hf = jnp.where(t < slc, nhf, hf)
        hb = jnp.where(TS - 1 - t < slc, nhb, hb)
        halls_ref[:, t, 0:HS] = hf
        halls_ref[:, TS - 1 - t, HS:2 * HS] = hb
    preps_ref[...] = _attend_rows(halls_ref[...], slc, saw, sab, sac)

    hf = jnp.zeros((NK, HP), jnp.float32)
    hb = jnp.zeros((NK, HP), jnp.float32)
    for t in range(TP):
        xf = preps_ref[t * NK:(t + 1) * NK]
        xb = preps_ref[(TP - 1 - t) * NK:(TP - t) * NK]
        gf = jnp.dot(xf, wpfx[...], preferred_element_type=jnp.float32) + bpfx[...]
        gb = jnp.dot(xb, wpbx[...], preferred_element_type=jnp.float32) + bpbx[...]
        ghf = jnp.dot(hf, wpfh[...], preferred_element_type=jnp.float32) + bpfh[...]
        ghb = jnp.dot(hb, wpbh[...], preferred_element_type=jnp.float32) + bpbh[...]
        nhf = _gru_gate(HP, gf, ghf, hf)
        nhb = _gru_gate(HP, gb, ghb, hb)
        hf = jnp.where(t < plc, nhf, hf)
        hb = jnp.where(TP - 1 - t < plc, nhb, hb)
        hallp_ref[:, t, 0:HP] = hf
        hallp_ref[:, TP - 1 - t, HP:2 * HP] = hb
    docrep = _attend_rows(hallp_ref[...], plc, paw, pab, pac)

    cur = docrep[0:8]
    prev = docrep[8:16]
    cat = jnp.concatenate([cur, prev, jnp.abs(cur - prev)], axis=1)
    h = jax.nn.relu(jnp.dot(cat, w1[...], preferred_element_type=jnp.float32)
                    + b1[...])
    out_ref[...] = jax.nn.sigmoid(
        jnp.dot(h, w2[...], preferred_element_type=jnp.float32) + b2[...])


def _upper(sx, slens, plens, ps_f, ps_b, saw, sab, sac,
           pp_f, pp_b, paw, pab, pac, w1, b1, w2, b2, wh, whp):
    HS, HP = 200, 300
    slc = jnp.broadcast_to(slens[:, None], (slens.shape[0], 8))
    plc = jnp.broadcast_to(plens[:, None], (plens.shape[0], 8))
    return pl.pallas_call(
        _upper_body,
        out_shape=jax.ShapeDtypeStruct((8, 1), jnp.float32),
        scratch_shapes=[
            pltpu.VMEM((64, 16, 3 * HS), jnp.float32),
            pltpu.VMEM((64, 16, 3 * HS), jnp.float32),
            pltpu.VMEM((64, 16, 2 * HS), jnp.float32),
            pltpu.VMEM((64, 2 * HS), jnp.float32),
            pltpu.VMEM((16, 4, 2 * HP), jnp.float32),
        ],
    )(sx, slc, plc,
      _pad_rows(ps_f['Wih'].T, wh, whp), ps_f['bih'].reshape(1, -1), ps_f['Whh'].T, ps_f['bhh'].reshape(1, -1),
      _pad_rows(ps_b['Wih'].T, wh, whp), ps_b['bih'].reshape(1, -1), ps_b['Whh'].T, ps_b['bhh'].reshape(1, -1),
      saw, sab.reshape(1, -1), sac.reshape(1, -1),
      pp_f['Wih'].T, pp_f['bih'].reshape(1, -1), pp_f['Whh'].T, pp_f['bhh'].reshape(1, -1),
      pp_b['Wih'].T, pp_b['bih'].reshape(1, -1), pp_b['Whh'].T, pp_b['bhh'].reshape(1, -1),
      paw, pab.reshape(1, -1), pac.reshape(1, -1),
      w1, b1.reshape(1, -1), w2, b2.reshape(1, -1))


def kernel(current_document, words_per_sentence_current_document,
           sentences_per_paragraph_current_document,
           paragraphs_per_document_current_document, previous_document,
           words_per_sentence_previous_document,
           sentences_per_paragraph_previous_document,
           paragraphs_per_document_previous_document, click_rate_tensor,
           params):
    p = params
    B, P, S, W = current_document.shape
    EMB = p['emb'].shape[1]

    nw = 2 * B * P * S
    ids = jnp.concatenate([current_document.reshape(-1),
                           previous_document.reshape(-1)]).astype(jnp.int32)
    ids_tm = ids.reshape(2, B, P, S, W).transpose(4, 2, 0, 1, 3).reshape(-1)
    emb = _emb_gather(p['emb'], ids_tm)

    x_w = emb.reshape(W, nw, EMB)
    wlens = jnp.concatenate([
        words_per_sentence_current_document.reshape(-1),
        words_per_sentence_previous_document.reshape(-1)]).astype(jnp.int32)
    wlens = wlens.reshape(2, B, P, S).transpose(2, 0, 1, 3).reshape(-1)
    WH = p['word_f']['Whh'].shape[1]
    WHP = 128
    sreps = _bigru_attend(x_w, wlens, p['word_f'], p['word_b'],
                          p['watt_W'], p['watt_b'], p['watt_c'],
                          nc=512, hp=WHP)

    sx = sreps.reshape(2 * B * P, S, sreps.shape[-1])
    slens = jnp.concatenate([
        sentences_per_paragraph_current_document.reshape(-1),
        sentences_per_paragraph_previous_document.reshape(-1)]).astype(jnp.int32)
    slens = slens.reshape(2, B, P).transpose(2, 0, 1).reshape(-1)
    plens = jnp.concatenate([
        paragraphs_per_document_current_document.reshape(-1),
        paragraphs_per_document_previous_document.reshape(-1)]).astype(jnp.int32)
    return _upper(sx, slens, plens, p['sent_f'], p['sent_b'],
                  p['satt_W'], p['satt_b'], p['satt_c'],
                  p['para_f'], p['para_b'],
                  p['patt_W'], p['patt_b'], p['patt_c'],
                  p['cls_W1'], p['cls_b1'], p['cls_W2'], p['cls_b2'],
                  WH, WHP)
